# baseline (speedup 1.0000x reference)
import math

import jax
import jax.numpy as jnp
from jax import lax
from jax.experimental import pallas as pl
from jax.experimental.pallas import tpu as pltpu
from jax.experimental.pallas import tpu_sc as plsc

B = 16384
V = 1000000
D = 16
NUC = 5
NIC = 3
H1 = 128
H2 = 64
EPS = 1e-3

NC = 2
NS = 16
NW = NC * NS
BPW = B // NW
CHUNK = 128
NCHUNK = BPW // CHUNK
F = (NUC + NIC) * D

BBLK = 2048


def _sc_gather(idx_all, tables):
  mesh = plsc.VectorSubcoreMesh(core_axis_name="c", subcore_axis_name="s",
                                num_cores=NC, num_subcores=NS)

  def body(idxp_hbm, idx_hbm, t0, t1, t2, t3, t4, t5, t6, t7,
           out_hbm, idxp_v, idx_v, pk_a, pk_b, feat_v, sem, semb):
    tbls = [t0, t1, t2, t3, t4, t5, t6, t7]
    lanes = lax.iota(jnp.int32, 16)
    wid = lax.axis_index("s") * NC + lax.axis_index("c")
    pks = [pk_a, pk_b]
    sems = [sem, semb]

    def fire(t, c, tw_unused=None):
      return pltpu.async_copy(
          tbls[t].at[idxp_v.at[t, pl.ds(c * CHUNK, CHUNK)]],
          pks[c % 2], sems[c % 2])

    def extract(t, c):
      def mgrp(m, _, t=t, c=c):
        v = idx_v[t, pl.ds(c * CHUNK + m * 16, 16)]
        s16 = (v & 7) * D
        rows = c * CHUNK + m * 16 + lanes
        for d in range(D):
          vals = plsc.load_gather(pks[c % 2], [m * 16 + lanes, s16 + d])
          plsc.store_scatter(feat_v, [rows, jnp.full((16,), t * D + d,
                                                     jnp.int32)], vals)
        return 0
      lax.fori_loop(0, CHUNK // 16, mgrp, 0)

    def tower(tw, _):
      pltpu.sync_copy(idxp_hbm.at[tw, wid], idxp_v)
      pltpu.sync_copy(idx_hbm.at[tw, wid], idx_v)
      for t in range(8):
        d0 = fire(t, 0)
        d1 = fire(t, 1)
        d0.wait()
        extract(t, 0)
        d2 = fire(t, 2)
        d1.wait()
        extract(t, 1)
        d3 = fire(t, 3)
        d2.wait()
        extract(t, 2)
        d3.wait()
        extract(t, 3)
      pltpu.sync_copy(feat_v, out_hbm.at[tw, pl.ds(wid * BPW, BPW)])
      return 0

    lax.fori_loop(0, 2, tower, 0)

  k = pl.kernel(
      body,
      out_type=jax.ShapeDtypeStruct((2, B, F), jnp.float32),
      mesh=mesh,
      scratch_types=[
          pltpu.VMEM((8, BPW), jnp.int32),
          pltpu.VMEM((8, BPW), jnp.int32),
          pltpu.VMEM((CHUNK, F), jnp.float32),
          pltpu.VMEM((CHUNK, F), jnp.float32),
          pltpu.VMEM((BPW, F), jnp.float32),
          pltpu.SemaphoreType.DMA,
          pltpu.SemaphoreType.DMA,
      ],
      compiler_params=pltpu.CompilerParams(needs_layout_passes=False),
  )
  return k(*idx_all, *tables)


def _dense_body(g_ref, un0_ref, in0_ref, un1_ref, in1_ref,
                wu1n_ref, wu1e_ref, bu1_ref, wu2_ref, bu2_ref,
                wi1n_ref, wi1e_ref, bi1_ref, wi2_ref, bi2_ref,
                wc1_ref, bc1_ref, wc2_ref, bc2_ref,
                wv1_ref, bv1_ref, wv2_ref, bv2_ref,
                ctr_ref, ctcvr_ref):
  s = 1.0 / math.sqrt(1.0 + EPS)

  emb_dot = lambda ft, w_ref: jnp.dot(
      ft, w_ref[:], preferred_element_type=jnp.float32)

  def tower(tw, un, inum, w1_ref, b1_ref, w2_ref, b2_ref):
    feat = g_ref[tw]
    hu = jnp.maximum(
        jnp.dot(un, wu1n_ref[:], preferred_element_type=jnp.float32)
        + emb_dot(feat, wu1e_ref)
        + bu1_ref[:], 0.0)
    uf = jnp.maximum(
        jnp.dot(hu, wu2_ref[:], preferred_element_type=jnp.float32)
        + bu2_ref[:], 0.0)
    hi = jnp.maximum(
        jnp.dot(inum, wi1n_ref[:], preferred_element_type=jnp.float32)
        + emb_dot(feat, wi1e_ref)
        + bi1_ref[:], 0.0)
    itf = jnp.maximum(
        jnp.dot(hi, wi2_ref[:], preferred_element_type=jnp.float32)
        + bi2_ref[:], 0.0)
    h = jnp.maximum(
        jnp.dot(uf * s, w1_ref[0:H2, :], preferred_element_type=jnp.float32)
        + jnp.dot(itf * s, w1_ref[H2:2 * H2, :],
                  preferred_element_type=jnp.float32)
        + b1_ref[:], 0.0)
    z = jnp.sum(h * w2_ref[:, 0], axis=1, keepdims=True) + b2_ref[:]
    return 1.0 / (1.0 + jnp.exp(-z))

  ctr = tower(0, un0_ref[:], in0_ref[:], wc1_ref, bc1_ref, wc2_ref, bc2_ref)
  cvr = tower(1, un1_ref[:], in1_ref[:], wv1_ref, bv1_ref, wv2_ref, bv2_ref)
  ctr_ref[:] = ctr
  ctcvr_ref[:] = ctr * cvr


def _tc_dense(g, un0, in0, un1, in1,
              wu1n, wu1e, bu1, wu2, bu2,
              wi1n, wi1e, bi1, wi2, bi2,
              wc1, bc1, wc2, bc2, wv1, bv1, wv2, bv2):
  grid = (B // BBLK,)
  full = lambda shape: pl.BlockSpec(shape, lambda i: tuple(0 for _ in shape))
  row = lambda shape: pl.BlockSpec(shape, lambda i: (i,) + (0,) * (len(shape) - 1))
  in_specs = [
      pl.BlockSpec((2, BBLK, F), lambda i: (0, i, 0)),
      row((BBLK, 8)), row((BBLK, 8)), row((BBLK, 8)), row((BBLK, 8)),
      full((8, H1)), full((F, H1)), full((H1,)), full((H1, H2)), full((H2,)),
      full((8, H1)), full((F, H1)), full((H1,)), full((H1, H2)), full((H2,)),
      full((2 * H2, H2)), full((H2,)), full((H2, 1)), full((1,)),
      full((2 * H2, H2)), full((H2,)), full((H2, 1)), full((1,)),
  ]
  out_specs = [row((BBLK, 1)), row((BBLK, 1))]
  out_shape = [jax.ShapeDtypeStruct((B, 1), jnp.float32)] * 2
  return pl.pallas_call(
      _dense_body,
      grid=grid,
      in_specs=in_specs,
      out_specs=out_specs,
      out_shape=out_shape,
      compiler_params=pltpu.CompilerParams(
          dimension_semantics=("parallel",)),
  )(g, un0, in0, un1, in1, wu1n, wu1e, bu1, wu2, bu2,
    wi1n, wi1e, bi1, wi2, bi2, wc1, bc1, wc2, bc2, wv1, bv1, wv2, bv2)


def kernel(ctr_user_numerical_input, ctr_user_cate_input,
           ctr_item_numerical_input, ctr_item_cate_input,
           cvr_user_numerical_input, cvr_user_cate_input,
           cvr_item_numerical_input, cvr_item_cate_input,
           embed_0, embed_1, embed_2, embed_3, embed_4, embed_5, embed_6,
           embed_7, Wu1, bu1, Wu2, bu2, Wi1, bi1, Wi2, bi2, Wc1, bc1, Wc2,
           bc2, Wv1, bv1, Wv2, bv2):
  ctr_idx = jnp.concatenate(
      [ctr_user_cate_input.T, ctr_item_cate_input.T], axis=0)
  cvr_idx = jnp.concatenate(
      [cvr_user_cate_input.T, cvr_item_cate_input.T], axis=0)
  idx_raw = jnp.stack([ctr_idx, cvr_idx]).astype(jnp.int32).reshape(
      2, 8, NW, BPW).transpose(0, 2, 1, 3)
  idx_pk = idx_raw >> 3

  tables = [t.reshape(V // 8, F) for t in
            (embed_0, embed_1, embed_2, embed_3, embed_4,
             embed_5, embed_6, embed_7)]
  g = _sc_gather((idx_pk, idx_raw), tables)

  pad_num = lambda x: jnp.pad(x, ((0, 0), (0, 3)))
  un0 = pad_num(ctr_user_numerical_input)
  in0 = pad_num(ctr_item_numerical_input)
  un1 = pad_num(cvr_user_numerical_input)
  in1 = pad_num(cvr_item_numerical_input)

  wu1n = jnp.pad(Wu1[:5], ((0, 3), (0, 0)))
  wu1e = jnp.pad(Wu1[5:], ((0, F - NUC * D), (0, 0)))
  wi1n = jnp.pad(Wi1[:5], ((0, 3), (0, 0)))
  wi1e = jnp.pad(Wi1[5:], ((NUC * D, 0), (0, 0)))

  ctr_pred, ctcvr_pred = _tc_dense(
      g, un0, in0, un1, in1,
      wu1n, wu1e, bu1, Wu2, bu2,
      wi1n, wi1e, bi1, Wi2, bi2,
      Wc1, bc1, Wc2, bc2, Wv1, bv1, Wv2, bv2)
  return ctr_pred, ctcvr_pred


# speedup vs baseline: 1.1857x; 1.1857x over previous
import math

import jax
import jax.numpy as jnp
from jax import lax
from jax.experimental import pallas as pl
from jax.experimental.pallas import tpu as pltpu
from jax.experimental.pallas import tpu_sc as plsc

B = 16384
V = 1000000
D = 16
NUC = 5
NIC = 3
H1 = 128
H2 = 64
EPS = 1e-3

NC = 2
NS = 16
NW = NC * NS
BPW = B // NW
CHUNK = 128
NCHUNK = BPW // CHUNK
F = (NUC + NIC) * D

BBLK = 2048


def _sc_gather(idx_all, tables):
  mesh = plsc.VectorSubcoreMesh(core_axis_name="c", subcore_axis_name="s",
                                num_cores=NC, num_subcores=NS)

  def body(idxp_hbm, idx_hbm, t0, t1, t2, t3, t4, t5, t6, t7,
           out_hbm, idxp_v, idx_v, pk_a, pk_b, feat_v, sem, semb):
    tbls = [t0, t1, t2, t3, t4, t5, t6, t7]
    lanes = lax.iota(jnp.int32, 16)
    wid = lax.axis_index("s") * NC + lax.axis_index("c")
    pks = [pk_a, pk_b]
    sems = [sem, semb]

    def fire(t, c, tw_unused=None):
      return pltpu.async_copy(
          tbls[t].at[idxp_v.at[t, pl.ds(c * CHUNK, CHUNK)]],
          pks[c % 2], sems[c % 2])

    def extract(t, c):
      def mgrp(m, _, t=t, c=c):
        v = idx_v[t, pl.ds(c * CHUNK + m * 16, 16)]
        s16 = (v & 7) * D
        rows = c * CHUNK + m * 16 + lanes
        for d in range(D):
          vals = plsc.load_gather(pks[c % 2], [m * 16 + lanes, s16 + d])
          plsc.store_scatter(feat_v, [rows, jnp.full((16,), t * D + d,
                                                     jnp.int32)], vals)
        return 0
      lax.fori_loop(0, CHUNK // 16, mgrp, 0)

    def tower(tw, _):
      pltpu.sync_copy(idxp_hbm.at[tw, wid], idxp_v)
      pltpu.sync_copy(idx_hbm.at[tw, wid], idx_v)
      for t in range(8):
        d0 = fire(t, 0)
        d1 = fire(t, 1)
        d0.wait()
        extract(t, 0)
        d2 = fire(t, 2)
        d1.wait()
        extract(t, 1)
        d3 = fire(t, 3)
        d2.wait()
        extract(t, 2)
        d3.wait()
        extract(t, 3)
      pltpu.sync_copy(feat_v, out_hbm.at[tw, pl.ds(wid * BPW, BPW)])
      return 0

    lax.fori_loop(0, 2, tower, 0)

  k = pl.kernel(
      body,
      out_type=jax.ShapeDtypeStruct((2, B, F), jnp.float32),
      mesh=mesh,
      scratch_types=[
          pltpu.VMEM((8, BPW), jnp.int32),
          pltpu.VMEM((8, BPW), jnp.int32),
          pltpu.VMEM((CHUNK, F), jnp.float32),
          pltpu.VMEM((CHUNK, F), jnp.float32),
          pltpu.VMEM((BPW, F), jnp.float32),
          pltpu.SemaphoreType.DMA,
          pltpu.SemaphoreType.DMA,
      ],
      compiler_params=pltpu.CompilerParams(needs_layout_passes=False),
  )
  return k(*idx_all, *tables)


CB = 4096


def _repack_body(*refs):
  ins, outs = refs[:8], refs[8:]
  for x_ref, z_ref in zip(ins, outs):
    y3 = x_ref[:].T.reshape(CB // 8, 8, D)
    z_ref[:] = jnp.concatenate([y3[:, s, :] for s in range(8)], axis=1)


def _tc_repack(tables_t):
  grid = (pl.cdiv(V, CB),)
  in_specs = [pl.BlockSpec((D, CB), lambda i: (0, i))] * 8
  out_specs = [pl.BlockSpec((CB // 8, F), lambda i: (i, 0))] * 8
  out_shape = [jax.ShapeDtypeStruct((V // 8, F), jnp.float32)] * 8
  return pl.pallas_call(
      _repack_body,
      grid=grid,
      in_specs=in_specs,
      out_specs=out_specs,
      out_shape=out_shape,
      compiler_params=pltpu.CompilerParams(
          dimension_semantics=("arbitrary",)),
  )(*tables_t)


def _dense_body(g_ref, un0_ref, in0_ref, un1_ref, in1_ref,
                wu1n_ref, wu1e_ref, bu1_ref, wu2_ref, bu2_ref,
                wi1n_ref, wi1e_ref, bi1_ref, wi2_ref, bi2_ref,
                wc1_ref, bc1_ref, wc2_ref, bc2_ref,
                wv1_ref, bv1_ref, wv2_ref, bv2_ref,
                ctr_ref, ctcvr_ref):
  s = 1.0 / math.sqrt(1.0 + EPS)

  emb_dot = lambda ft, w_ref: jnp.dot(
      ft, w_ref[:], preferred_element_type=jnp.float32)

  def tower(tw, un, inum, w1_ref, b1_ref, w2_ref, b2_ref):
    feat = g_ref[tw]
    hu = jnp.maximum(
        jnp.dot(un, wu1n_ref[:], preferred_element_type=jnp.float32)
        + emb_dot(feat, wu1e_ref)
        + bu1_ref[:], 0.0)
    uf = jnp.maximum(
        jnp.dot(hu, wu2_ref[:], preferred_element_type=jnp.float32)
        + bu2_ref[:], 0.0)
    hi = jnp.maximum(
        jnp.dot(inum, wi1n_ref[:], preferred_element_type=jnp.float32)
        + emb_dot(feat, wi1e_ref)
        + bi1_ref[:], 0.0)
    itf = jnp.maximum(
        jnp.dot(hi, wi2_ref[:], preferred_element_type=jnp.float32)
        + bi2_ref[:], 0.0)
    h = jnp.maximum(
        jnp.dot(uf * s, w1_ref[0:H2, :], preferred_element_type=jnp.float32)
        + jnp.dot(itf * s, w1_ref[H2:2 * H2, :],
                  preferred_element_type=jnp.float32)
        + b1_ref[:], 0.0)
    z = jnp.sum(h * w2_ref[:, 0], axis=1, keepdims=True) + b2_ref[:]
    return 1.0 / (1.0 + jnp.exp(-z))

  ctr = tower(0, un0_ref[:], in0_ref[:], wc1_ref, bc1_ref, wc2_ref, bc2_ref)
  cvr = tower(1, un1_ref[:], in1_ref[:], wv1_ref, bv1_ref, wv2_ref, bv2_ref)
  ctr_ref[:] = ctr
  ctcvr_ref[:] = ctr * cvr


def _tc_dense(g, un0, in0, un1, in1,
              wu1n, wu1e, bu1, wu2, bu2,
              wi1n, wi1e, bi1, wi2, bi2,
              wc1, bc1, wc2, bc2, wv1, bv1, wv2, bv2):
  grid = (B // BBLK,)
  full = lambda shape: pl.BlockSpec(shape, lambda i: tuple(0 for _ in shape))
  row = lambda shape: pl.BlockSpec(shape, lambda i: (i,) + (0,) * (len(shape) - 1))
  in_specs = [
      pl.BlockSpec((2, BBLK, F), lambda i: (0, i, 0)),
      row((BBLK, 8)), row((BBLK, 8)), row((BBLK, 8)), row((BBLK, 8)),
      full((8, H1)), full((F, H1)), full((H1,)), full((H1, H2)), full((H2,)),
      full((8, H1)), full((F, H1)), full((H1,)), full((H1, H2)), full((H2,)),
      full((2 * H2, H2)), full((H2,)), full((H2, 1)), full((1,)),
      full((2 * H2, H2)), full((H2,)), full((H2, 1)), full((1,)),
  ]
  out_specs = [row((BBLK, 1)), row((BBLK, 1))]
  out_shape = [jax.ShapeDtypeStruct((B, 1), jnp.float32)] * 2
  return pl.pallas_call(
      _dense_body,
      grid=grid,
      in_specs=in_specs,
      out_specs=out_specs,
      out_shape=out_shape,
      compiler_params=pltpu.CompilerParams(
          dimension_semantics=("parallel",)),
  )(g, un0, in0, un1, in1, wu1n, wu1e, bu1, wu2, bu2,
    wi1n, wi1e, bi1, wi2, bi2, wc1, bc1, wc2, bc2, wv1, bv1, wv2, bv2)


def kernel(ctr_user_numerical_input, ctr_user_cate_input,
           ctr_item_numerical_input, ctr_item_cate_input,
           cvr_user_numerical_input, cvr_user_cate_input,
           cvr_item_numerical_input, cvr_item_cate_input,
           embed_0, embed_1, embed_2, embed_3, embed_4, embed_5, embed_6,
           embed_7, Wu1, bu1, Wu2, bu2, Wi1, bi1, Wi2, bi2, Wc1, bc1, Wc2,
           bc2, Wv1, bv1, Wv2, bv2):
  ctr_idx = jnp.concatenate(
      [ctr_user_cate_input.T, ctr_item_cate_input.T], axis=0)
  cvr_idx = jnp.concatenate(
      [cvr_user_cate_input.T, cvr_item_cate_input.T], axis=0)
  idx_raw = jnp.stack([ctr_idx, cvr_idx]).astype(jnp.int32).reshape(
      2, 8, NW, BPW).transpose(0, 2, 1, 3)
  idx_pk = idx_raw >> 3

  tables = _tc_repack([t.T for t in
                       (embed_0, embed_1, embed_2, embed_3, embed_4,
                        embed_5, embed_6, embed_7)])
  g = _sc_gather((idx_pk, idx_raw), tables)

  pad_num = lambda x: jnp.pad(x, ((0, 0), (0, 3)))
  un0 = pad_num(ctr_user_numerical_input)
  in0 = pad_num(ctr_item_numerical_input)
  un1 = pad_num(cvr_user_numerical_input)
  in1 = pad_num(cvr_item_numerical_input)

  wu1n = jnp.pad(Wu1[:5], ((0, 3), (0, 0)))
  wu1e = jnp.pad(Wu1[5:], ((0, F - NUC * D), (0, 0)))
  wi1n = jnp.pad(Wi1[:5], ((0, 3), (0, 0)))
  wi1e = jnp.pad(Wi1[5:], ((NUC * D, 0), (0, 0)))

  ctr_pred, ctcvr_pred = _tc_dense(
      g, un0, in0, un1, in1,
      wu1n, wu1e, bu1, Wu2, bu2,
      wi1n, wi1e, bi1, Wi2, bi2,
      Wc1, bc1, Wc2, bc2, Wv1, bv1, Wv2, bv2)
  return ctr_pred, ctcvr_pred


# speedup vs baseline: 1.3406x; 1.1306x over previous
import math

import jax
import jax.numpy as jnp
from jax import lax
from jax.experimental import pallas as pl
from jax.experimental.pallas import tpu as pltpu
from jax.experimental.pallas import tpu_sc as plsc

B = 16384
V = 1000000
D = 16
NUC = 5
NIC = 3
H1 = 128
H2 = 64
EPS = 1e-3

NC = 2
NS = 16
NW = NC * NS
BPW = B // NW
CHUNK = 128
NCHUNK = BPW // CHUNK
F = (NUC + NIC) * D

BBLK = 2048


def _sc_gather(idx_all, tables):
  mesh = plsc.VectorSubcoreMesh(core_axis_name="c", subcore_axis_name="s",
                                num_cores=NC, num_subcores=NS)

  def body(idxp_hbm, idx_hbm, t0, t1, t2, t3, t4, t5, t6, t7,
           out_hbm, idxp_v, idx_v, pk_a, pk_b, feat_v, sem, semb):
    tbls = [t0, t1, t2, t3, t4, t5, t6, t7]
    lanes = lax.iota(jnp.int32, 16)
    wid = lax.axis_index("s") * NC + lax.axis_index("c")
    pks = [pk_a, pk_b]
    sems = [sem, semb]

    def fire(t, c, tw_unused=None):
      return pltpu.async_copy(
          tbls[t].at[idxp_v.at[t, pl.ds(c * CHUNK, CHUNK)]],
          pks[c % 2], sems[c % 2])

    def extract(t, c):
      def mgrp(m, _, t=t, c=c):
        v = idx_v[t, pl.ds(c * CHUNK + m * 16, 16)]
        s16 = (v & 7) * D
        rows = c * CHUNK + m * 16 + lanes
        for d in range(D):
          vals = plsc.load_gather(pks[c % 2], [m * 16 + lanes, s16 + d])
          plsc.store_scatter(feat_v, [rows, jnp.full((16,), t * D + d,
                                                     jnp.int32)], vals)
        return 0
      lax.fori_loop(0, CHUNK // 16, mgrp, 0)

    def tower(tw, _):
      pltpu.sync_copy(idxp_hbm.at[tw, wid], idxp_v)
      pltpu.sync_copy(idx_hbm.at[tw, wid], idx_v)
      for t in range(8):
        d0 = fire(t, 0)
        d1 = fire(t, 1)
        d0.wait()
        extract(t, 0)
        d2 = fire(t, 2)
        d1.wait()
        extract(t, 1)
        d3 = fire(t, 3)
        d2.wait()
        extract(t, 2)
        d3.wait()
        extract(t, 3)
      pltpu.sync_copy(feat_v, out_hbm.at[tw, pl.ds(wid * BPW, BPW)])
      return 0

    lax.fori_loop(0, 2, tower, 0)

  k = pl.kernel(
      body,
      out_type=jax.ShapeDtypeStruct((2, B, F), jnp.float32),
      mesh=mesh,
      scratch_types=[
          pltpu.VMEM((8, BPW), jnp.int32),
          pltpu.VMEM((8, BPW), jnp.int32),
          pltpu.VMEM((CHUNK, F), jnp.float32),
          pltpu.VMEM((CHUNK, F), jnp.float32),
          pltpu.VMEM((BPW, F), jnp.float32),
          pltpu.SemaphoreType.DMA,
          pltpu.SemaphoreType.DMA,
      ],
      compiler_params=pltpu.CompilerParams(needs_layout_passes=False),
  )
  return k(*idx_all, *tables)


CB = 4096


def _repack_body(*refs):
  ins, outs = refs[:8], refs[8:]
  for x_ref, z_ref in zip(ins, outs):
    y3 = x_ref[:].T.reshape(CB // 8, 8, D)
    for s in range(8):
      z_ref[:, s * D:(s + 1) * D] = y3[:, s, :]


def _tc_repack(tables_t):
  grid = (pl.cdiv(V, CB),)
  in_specs = [pl.BlockSpec((D, CB), lambda i: (0, i))] * 8
  out_specs = [pl.BlockSpec((CB // 8, F), lambda i: (i, 0))] * 8
  out_shape = [jax.ShapeDtypeStruct((V // 8, F), jnp.float32)] * 8
  return pl.pallas_call(
      _repack_body,
      grid=grid,
      in_specs=in_specs,
      out_specs=out_specs,
      out_shape=out_shape,
      compiler_params=pltpu.CompilerParams(
          dimension_semantics=("arbitrary",)),
  )(*tables_t)


def _dense_body(g_ref, un0_ref, in0_ref, un1_ref, in1_ref,
                wu1n_ref, wu1e_ref, bu1_ref, wu2_ref, bu2_ref,
                wi1n_ref, wi1e_ref, bi1_ref, wi2_ref, bi2_ref,
                wc1_ref, bc1_ref, wc2_ref, bc2_ref,
                wv1_ref, bv1_ref, wv2_ref, bv2_ref,
                ctr_ref, ctcvr_ref):
  s = 1.0 / math.sqrt(1.0 + EPS)

  emb_dot = lambda ft, w_ref: jnp.dot(
      ft, w_ref[:], preferred_element_type=jnp.float32)

  def tower(tw, un, inum, w1_ref, b1_ref, w2_ref, b2_ref):
    feat = g_ref[tw]
    hu = jnp.maximum(
        jnp.dot(un, wu1n_ref[:], preferred_element_type=jnp.float32)
        + emb_dot(feat, wu1e_ref)
        + bu1_ref[:], 0.0)
    uf = jnp.maximum(
        jnp.dot(hu, wu2_ref[:], preferred_element_type=jnp.float32)
        + bu2_ref[:], 0.0)
    hi = jnp.maximum(
        jnp.dot(inum, wi1n_ref[:], preferred_element_type=jnp.float32)
        + emb_dot(feat, wi1e_ref)
        + bi1_ref[:], 0.0)
    itf = jnp.maximum(
        jnp.dot(hi, wi2_ref[:], preferred_element_type=jnp.float32)
        + bi2_ref[:], 0.0)
    h = jnp.maximum(
        jnp.dot(uf * s, w1_ref[0:H2, :], preferred_element_type=jnp.float32)
        + jnp.dot(itf * s, w1_ref[H2:2 * H2, :],
                  preferred_element_type=jnp.float32)
        + b1_ref[:], 0.0)
    z = jnp.sum(h * w2_ref[:, 0], axis=1, keepdims=True) + b2_ref[:]
    return 1.0 / (1.0 + jnp.exp(-z))

  ctr = tower(0, un0_ref[:], in0_ref[:], wc1_ref, bc1_ref, wc2_ref, bc2_ref)
  cvr = tower(1, un1_ref[:], in1_ref[:], wv1_ref, bv1_ref, wv2_ref, bv2_ref)
  ctr_ref[:] = ctr
  ctcvr_ref[:] = ctr * cvr


def _tc_dense(g, un0, in0, un1, in1,
              wu1n, wu1e, bu1, wu2, bu2,
              wi1n, wi1e, bi1, wi2, bi2,
              wc1, bc1, wc2, bc2, wv1, bv1, wv2, bv2):
  grid = (B // BBLK,)
  full = lambda shape: pl.BlockSpec(shape, lambda i: tuple(0 for _ in shape))
  row = lambda shape: pl.BlockSpec(shape, lambda i: (i,) + (0,) * (len(shape) - 1))
  in_specs = [
      pl.BlockSpec((2, BBLK, F), lambda i: (0, i, 0)),
      row((BBLK, 8)), row((BBLK, 8)), row((BBLK, 8)), row((BBLK, 8)),
      full((8, H1)), full((F, H1)), full((H1,)), full((H1, H2)), full((H2,)),
      full((8, H1)), full((F, H1)), full((H1,)), full((H1, H2)), full((H2,)),
      full((2 * H2, H2)), full((H2,)), full((H2, 1)), full((1,)),
      full((2 * H2, H2)), full((H2,)), full((H2, 1)), full((1,)),
  ]
  out_specs = [row((BBLK, 1)), row((BBLK, 1))]
  out_shape = [jax.ShapeDtypeStruct((B, 1), jnp.float32)] * 2
  return pl.pallas_call(
      _dense_body,
      grid=grid,
      in_specs=in_specs,
      out_specs=out_specs,
      out_shape=out_shape,
      compiler_params=pltpu.CompilerParams(
          dimension_semantics=("parallel",)),
  )(g, un0, in0, un1, in1, wu1n, wu1e, bu1, wu2, bu2,
    wi1n, wi1e, bi1, wi2, bi2, wc1, bc1, wc2, bc2, wv1, bv1, wv2, bv2)


def kernel(ctr_user_numerical_input, ctr_user_cate_input,
           ctr_item_numerical_input, ctr_item_cate_input,
           cvr_user_numerical_input, cvr_user_cate_input,
           cvr_item_numerical_input, cvr_item_cate_input,
           embed_0, embed_1, embed_2, embed_3, embed_4, embed_5, embed_6,
           embed_7, Wu1, bu1, Wu2, bu2, Wi1, bi1, Wi2, bi2, Wc1, bc1, Wc2,
           bc2, Wv1, bv1, Wv2, bv2):
  ctr_idx = jnp.concatenate(
      [ctr_user_cate_input.T, ctr_item_cate_input.T], axis=0)
  cvr_idx = jnp.concatenate(
      [cvr_user_cate_input.T, cvr_item_cate_input.T], axis=0)
  idx_raw = jnp.stack([ctr_idx, cvr_idx]).astype(jnp.int32).reshape(
      2, 8, NW, BPW).transpose(0, 2, 1, 3)
  idx_pk = idx_raw >> 3

  tables = _tc_repack([t.T for t in
                       (embed_0, embed_1, embed_2, embed_3, embed_4,
                        embed_5, embed_6, embed_7)])
  g = _sc_gather((idx_pk, idx_raw), tables)

  pad_num = lambda x: jnp.pad(x, ((0, 0), (0, 3)))
  un0 = pad_num(ctr_user_numerical_input)
  in0 = pad_num(ctr_item_numerical_input)
  un1 = pad_num(cvr_user_numerical_input)
  in1 = pad_num(cvr_item_numerical_input)

  wu1n = jnp.pad(Wu1[:5], ((0, 3), (0, 0)))
  wu1e = jnp.pad(Wu1[5:], ((0, F - NUC * D), (0, 0)))
  wi1n = jnp.pad(Wi1[:5], ((0, 3), (0, 0)))
  wi1e = jnp.pad(Wi1[5:], ((NUC * D, 0), (0, 0)))

  ctr_pred, ctcvr_pred = _tc_dense(
      g, un0, in0, un1, in1,
      wu1n, wu1e, bu1, Wu2, bu2,
      wi1n, wi1e, bi1, Wi2, bi2,
      Wc1, bc1, Wc2, bc2, Wv1, bv1, Wv2, bv2)
  return ctr_pred, ctcvr_pred


# speedup vs baseline: 2.6199x; 1.9543x over previous
import math

import jax
import jax.numpy as jnp
from jax import lax
from jax.experimental import pallas as pl
from jax.experimental.pallas import tpu as pltpu
from jax.experimental.pallas import tpu_sc as plsc

B = 16384
V = 1000000
D = 16
NUC = 5
NIC = 3
H1 = 128
H2 = 64
EPS = 1e-3

NC = 2
NS = 16
NW = NC * NS
BPW = B // NW
CHUNK = 128
NCHUNK = BPW // CHUNK
F = (NUC + NIC) * D

BBLK = 2048


def _sc_gather(idx_all, tables):
  mesh = plsc.VectorSubcoreMesh(core_axis_name="c", subcore_axis_name="s",
                                num_cores=NC, num_subcores=NS)

  def body(idx_hbm, t0, t1, t2, t3, t4, t5, t6, t7,
           out_hbm, idx_v, feat_v, sem):
    tbls = [t0, t1, t2, t3, t4, t5, t6, t7]
    lanes = lax.iota(jnp.int32, 16)
    wid = lax.axis_index("s") * NC + lax.axis_index("c")
    for tw in range(2):
      pltpu.sync_copy(idx_hbm.at[tw, wid], idx_v)
      for t in range(8):

        def grp(g, _, t=t):
          v = idx_v[t, pl.ds(g * 16, 16)]
          for u in range(16):
            i = jnp.sum(jnp.where(lanes == u, v, 0))
            pltpu.async_copy(tbls[t].at[i >> 3, i & 7],
                             feat_v.at[g * 16 + u, pl.ds(t * D, D)], sem)
          return 0

        lax.fori_loop(0, BPW // 16, grp, 0)
      pltpu.make_async_copy(out_hbm.at[tw, pl.ds(wid * BPW, BPW)],
                            feat_v, sem).wait()
      pltpu.sync_copy(feat_v, out_hbm.at[tw, pl.ds(wid * BPW, BPW)])

  k = pl.kernel(
      body,
      out_type=jax.ShapeDtypeStruct((2, B, F), jnp.float32),
      mesh=mesh,
      scratch_types=[
          pltpu.VMEM((8, BPW), jnp.int32),
          pltpu.VMEM((BPW, F), jnp.float32),
          pltpu.SemaphoreType.DMA,
      ],
      compiler_params=pltpu.CompilerParams(needs_layout_passes=False),
  )
  return k(idx_all, *tables)


def _dense_body(g_ref, un0_ref, in0_ref, un1_ref, in1_ref,
                wu1n_ref, wu1e_ref, bu1_ref, wu2_ref, bu2_ref,
                wi1n_ref, wi1e_ref, bi1_ref, wi2_ref, bi2_ref,
                wc1_ref, bc1_ref, wc2_ref, bc2_ref,
                wv1_ref, bv1_ref, wv2_ref, bv2_ref,
                ctr_ref, ctcvr_ref):
  s = 1.0 / math.sqrt(1.0 + EPS)

  emb_dot = lambda ft, w_ref: jnp.dot(
      ft, w_ref[:], preferred_element_type=jnp.float32)

  def tower(tw, un, inum, w1_ref, b1_ref, w2_ref, b2_ref):
    feat = g_ref[tw]
    hu = jnp.maximum(
        jnp.dot(un, wu1n_ref[:], preferred_element_type=jnp.float32)
        + emb_dot(feat, wu1e_ref)
        + bu1_ref[:], 0.0)
    uf = jnp.maximum(
        jnp.dot(hu, wu2_ref[:], preferred_element_type=jnp.float32)
        + bu2_ref[:], 0.0)
    hi = jnp.maximum(
        jnp.dot(inum, wi1n_ref[:], preferred_element_type=jnp.float32)
        + emb_dot(feat, wi1e_ref)
        + bi1_ref[:], 0.0)
    itf = jnp.maximum(
        jnp.dot(hi, wi2_ref[:], preferred_element_type=jnp.float32)
        + bi2_ref[:], 0.0)
    h = jnp.maximum(
        jnp.dot(uf * s, w1_ref[0:H2, :], preferred_element_type=jnp.float32)
        + jnp.dot(itf * s, w1_ref[H2:2 * H2, :],
                  preferred_element_type=jnp.float32)
        + b1_ref[:], 0.0)
    z = jnp.sum(h * w2_ref[:, 0], axis=1, keepdims=True) + b2_ref[:]
    return 1.0 / (1.0 + jnp.exp(-z))

  ctr = tower(0, un0_ref[:], in0_ref[:], wc1_ref, bc1_ref, wc2_ref, bc2_ref)
  cvr = tower(1, un1_ref[:], in1_ref[:], wv1_ref, bv1_ref, wv2_ref, bv2_ref)
  ctr_ref[:] = ctr
  ctcvr_ref[:] = ctr * cvr


def _tc_dense(g, un0, in0, un1, in1,
              wu1n, wu1e, bu1, wu2, bu2,
              wi1n, wi1e, bi1, wi2, bi2,
              wc1, bc1, wc2, bc2, wv1, bv1, wv2, bv2):
  grid = (B // BBLK,)
  full = lambda shape: pl.BlockSpec(shape, lambda i: tuple(0 for _ in shape))
  row = lambda shape: pl.BlockSpec(shape, lambda i: (i,) + (0,) * (len(shape) - 1))
  in_specs = [
      pl.BlockSpec((2, BBLK, F), lambda i: (0, i, 0)),
      row((BBLK, 8)), row((BBLK, 8)), row((BBLK, 8)), row((BBLK, 8)),
      full((8, H1)), full((F, H1)), full((H1,)), full((H1, H2)), full((H2,)),
      full((8, H1)), full((F, H1)), full((H1,)), full((H1, H2)), full((H2,)),
      full((2 * H2, H2)), full((H2,)), full((H2, 1)), full((1,)),
      full((2 * H2, H2)), full((H2,)), full((H2, 1)), full((1,)),
  ]
  out_specs = [row((BBLK, 1)), row((BBLK, 1))]
  out_shape = [jax.ShapeDtypeStruct((B, 1), jnp.float32)] * 2
  return pl.pallas_call(
      _dense_body,
      grid=grid,
      in_specs=in_specs,
      out_specs=out_specs,
      out_shape=out_shape,
      compiler_params=pltpu.CompilerParams(
          dimension_semantics=("parallel",)),
  )(g, un0, in0, un1, in1, wu1n, wu1e, bu1, wu2, bu2,
    wi1n, wi1e, bi1, wi2, bi2, wc1, bc1, wc2, bc2, wv1, bv1, wv2, bv2)


def kernel(ctr_user_numerical_input, ctr_user_cate_input,
           ctr_item_numerical_input, ctr_item_cate_input,
           cvr_user_numerical_input, cvr_user_cate_input,
           cvr_item_numerical_input, cvr_item_cate_input,
           embed_0, embed_1, embed_2, embed_3, embed_4, embed_5, embed_6,
           embed_7, Wu1, bu1, Wu2, bu2, Wi1, bi1, Wi2, bi2, Wc1, bc1, Wc2,
           bc2, Wv1, bv1, Wv2, bv2):
  ctr_idx = jnp.concatenate(
      [ctr_user_cate_input.T, ctr_item_cate_input.T], axis=0)
  cvr_idx = jnp.concatenate(
      [cvr_user_cate_input.T, cvr_item_cate_input.T], axis=0)
  idx_all = jnp.stack([ctr_idx, cvr_idx]).astype(jnp.int32).reshape(
      2, 8, NW, BPW).transpose(0, 2, 1, 3)

  tables = [t.reshape(V // 8, 8, D) for t in
            (embed_0, embed_1, embed_2, embed_3, embed_4,
             embed_5, embed_6, embed_7)]
  g = _sc_gather(idx_all, tables)

  pad_num = lambda x: jnp.pad(x, ((0, 0), (0, 3)))
  un0 = pad_num(ctr_user_numerical_input)
  in0 = pad_num(ctr_item_numerical_input)
  un1 = pad_num(cvr_user_numerical_input)
  in1 = pad_num(cvr_item_numerical_input)

  wu1n = jnp.pad(Wu1[:5], ((0, 3), (0, 0)))
  wu1e = jnp.pad(Wu1[5:], ((0, F - NUC * D), (0, 0)))
  wi1n = jnp.pad(Wi1[:5], ((0, 3), (0, 0)))
  wi1e = jnp.pad(Wi1[5:], ((NUC * D, 0), (0, 0)))

  ctr_pred, ctcvr_pred = _tc_dense(
      g, un0, in0, un1, in1,
      wu1n, wu1e, bu1, Wu2, bu2,
      wi1n, wi1e, bi1, Wi2, bi2,
      Wc1, bc1, Wc2, bc2, Wv1, bv1, Wv2, bv2)
  return ctr_pred, ctcvr_pred
